# baseline (speedup 1.0000x reference)
import functools

import jax
import jax.numpy as jnp
from jax import lax
from jax.experimental import pallas as pl
from jax.experimental.pallas import tpu as pltpu
from jax.experimental.pallas import tpu_sc as plsc

NUM_CORES = 2
NUM_SUBCORES = 16
LANES = 16

CHUNK = 64
GROUP = 16
ZROWS = 16
SUBROWS = 624


def _propagate(x, src, dst):
    n, d = x.shape
    e = src.shape[0]
    nworkers = NUM_CORES * NUM_SUBCORES
    ngroups = -(-e // (nworkers * GROUP * CHUNK))
    nchunk = ngroups * GROUP
    epw = nchunk * CHUNK
    pad = nworkers * epw - e
    npad = n + 8
    tail = n - SUBROWS * NUM_SUBCORES

    mesh = plsc.VectorSubcoreMesh(core_axis_name="c", subcore_axis_name="s")

    @functools.partial(
        pl.kernel,
        mesh=mesh,
        out_type=jax.ShapeDtypeStruct((NUM_CORES, n, d), jnp.float32),
        scratch_types=[
            pltpu.VMEM_SHARED((npad, d), jnp.float32),
            pltpu.VMEM((2, GROUP, CHUNK), jnp.int32),
            pltpu.VMEM((2, GROUP, CHUNK), jnp.int32),
            pltpu.VMEM((2, CHUNK, d), jnp.float32),
            pltpu.SemaphoreType.DMA((2,)),
            pltpu.SemaphoreType.DMA((2,)),
            pltpu.SemaphoreType.DMA((2,)),
        ],
    )
    def prop(x_hbm, src_hbm, dst_hbm, out_hbm, acc, sidx, didx, rows,
             gsem, ssem, isem):
        cid = lax.axis_index("c")
        sid = lax.axis_index("s")
        wid = cid * NUM_SUBCORES + sid

        zero = jnp.zeros((LANES,), jnp.float32)

        def zfill(i, _):
            r = i // (d // LANES)
            col = (i % (d // LANES)) * LANES
            rows[0, r, pl.ds(col, LANES)] = zero
            return 0

        lax.fori_loop(0, ZROWS * (d // LANES), zfill, 0)
        zsrc = rows.at[0, pl.ds(0, ZROWS)]

        def zcopy(j, _):
            pltpu.sync_copy(
                zsrc, acc.at[pl.ds(sid * SUBROWS + j * ZROWS, ZROWS)])
            return 0

        lax.fori_loop(0, SUBROWS // ZROWS, zcopy, 0)

        @pl.when(sid == NUM_SUBCORES - 1)
        def _():
            def ztail(j, _):
                pltpu.sync_copy(
                    zsrc,
                    acc.at[pl.ds(NUM_SUBCORES * SUBROWS + j * ZROWS, ZROWS)])
                return 0
            lax.fori_loop(0, tail // ZROWS, ztail, 0)

        pltpu.sync_copy(src_hbm.at[wid, 0], sidx.at[0])
        pltpu.sync_copy(dst_hbm.at[wid, 0], didx.at[0])
        plsc.subcore_barrier()

        def idx_of(ci):
            return lax.rem(ci // GROUP, 2), lax.rem(ci, GROUP)

        pltpu.async_copy(x_hbm.at[sidx.at[0, 0]], rows.at[0], gsem.at[0])

        def body(ci, _):
            b = lax.rem(ci, 2)
            nb = 1 - b
            g = ci // GROUP
            s = lax.rem(ci, GROUP)
            gb = lax.rem(g, 2)
            ngb = 1 - gb
            ib, is_ = idx_of(ci)

            @pl.when(jnp.logical_and(ci >= 1, ci + 1 < nchunk))
            def _():
                pib, pis = idx_of(ci - 1)
                pltpu.make_async_copy(rows.at[nb], acc.at[didx.at[pib, pis]],
                                      ssem.at[nb]).wait()

            @pl.when(jnp.logical_and(s == 0, (g + 1) * GROUP < nchunk))
            def _():
                pltpu.async_copy(src_hbm.at[wid, g + 1], sidx.at[ngb],
                                 isem.at[ngb])
                pltpu.async_copy(dst_hbm.at[wid, g + 1], didx.at[ngb],
                                 isem.at[ngb])

            @pl.when(jnp.logical_and(s == GROUP - 1, ci + 1 < nchunk))
            def _():
                pltpu.make_async_copy(src_hbm.at[wid, g + 1], sidx.at[ngb],
                                      isem.at[ngb]).wait()
                pltpu.make_async_copy(dst_hbm.at[wid, g + 1], didx.at[ngb],
                                      isem.at[ngb]).wait()

            @pl.when(ci + 1 < nchunk)
            def _():
                fib, fis = idx_of(ci + 1)
                pltpu.async_copy(x_hbm.at[sidx.at[fib, fis]], rows.at[nb],
                                 gsem.at[nb])

            pltpu.make_async_copy(x_hbm.at[sidx.at[ib, is_]], rows.at[b],
                                  gsem.at[b]).wait()
            pltpu.async_copy(rows.at[b], acc.at[didx.at[ib, is_]], ssem.at[b],
                             add=True)
            return 0

        lax.fori_loop(0, nchunk, body, 0)
        lastb = (nchunk - 1) % 2
        lib, lis = idx_of(nchunk - 2)
        pltpu.make_async_copy(rows.at[1 - lastb], acc.at[didx.at[lib, lis]],
                              ssem.at[1 - lastb]).wait()
        lib, lis = idx_of(nchunk - 1)
        pltpu.make_async_copy(rows.at[lastb], acc.at[didx.at[lib, lis]],
                              ssem.at[lastb]).wait()
        plsc.subcore_barrier()

        rbase = sid * SUBROWS
        pltpu.sync_copy(acc.at[pl.ds(rbase, SUBROWS)],
                        out_hbm.at[cid, pl.ds(rbase, SUBROWS)])

        @pl.when(sid == NUM_SUBCORES - 1)
        def _():
            tbase = NUM_SUBCORES * SUBROWS
            pltpu.sync_copy(acc.at[pl.ds(tbase, tail)],
                            out_hbm.at[cid, pl.ds(tbase, tail)])

    if pad:
        src = jnp.concatenate([src, jnp.zeros((pad,), jnp.int32)])
        dst = jnp.concatenate([dst, jnp.full((pad,), n, jnp.int32)])
    src4 = src.reshape(nworkers, ngroups, GROUP, CHUNK)
    dst4 = dst.reshape(nworkers, ngroups, GROUP, CHUNK)
    return prop(x, src4, dst4)


def _dense(p, w, block, final):
    n = p.shape[1]
    d = p.shape[2]

    def body(p_ref, w_ref, o_ref):
        h = p_ref[0] + p_ref[1]
        z = lax.dot_general(h, w_ref[...], (((1,), (1,)), ((), ())),
                            preferred_element_type=jnp.float32)
        if final:
            m = jnp.max(z, axis=1, keepdims=True)
            s = z - m
            lse = jnp.log(jnp.sum(jnp.exp(s), axis=1, keepdims=True))
            o_ref[...] = s - lse
        else:
            o_ref[...] = jnp.maximum(z, 0.0)

    return pl.pallas_call(
        body,
        grid=(n // block,),
        in_specs=[
            pl.BlockSpec((NUM_CORES, block, d), lambda i: (0, i, 0)),
            pl.BlockSpec((d, d), lambda i: (0, 0)),
        ],
        out_specs=pl.BlockSpec((block, d), lambda i: (i, 0)),
        out_shape=jax.ShapeDtypeStruct((n, d), jnp.float32),
    )(p, w)


def kernel(x, edge_index, W1, W2):
    src = edge_index[0]
    dst = edge_index[1]
    p = _propagate(x, src, dst)
    h = _dense(p, W1, 1000, final=False)
    q = _propagate(h, src, dst)
    return _dense(q, W2, 1000, final=True)


# speedup vs baseline: 1.9072x; 1.9072x over previous
import functools

import jax
import jax.numpy as jnp
from jax import lax
from jax.experimental import pallas as pl
from jax.experimental.pallas import tpu as pltpu
from jax.experimental.pallas import tpu_sc as plsc

NUM_CORES = 2
NUM_SUBCORES = 16
LANES = 16

CHUNK = 96
GROUP = 21
ZROWS = 16
SUBROWS = 624
NWORKERS = NUM_CORES * NUM_SUBCORES


def _pad_edges(src, dst, n):
    e = src.shape[0]
    ngroups = -(-e // (NWORKERS * GROUP * CHUNK))
    pad = NWORKERS * ngroups * GROUP * CHUNK - e
    if pad:
        src = jnp.concatenate([src, jnp.zeros((pad,), jnp.int32)])
        dst = jnp.concatenate([dst, jnp.full((pad,), n, jnp.int32)])
    shape = (NWORKERS, ngroups, GROUP, CHUNK)
    return src.reshape(shape), dst.reshape(shape)


def _propagate(x, src4, dst4):
    n, d = x.shape
    ngroups = src4.shape[1]
    npad = n + 8
    tail = n - SUBROWS * NUM_SUBCORES

    mesh = plsc.VectorSubcoreMesh(core_axis_name="c", subcore_axis_name="s")

    @functools.partial(
        pl.kernel,
        mesh=mesh,
        out_type=jax.ShapeDtypeStruct((NUM_CORES, n, d), jnp.float32),
        scratch_types=[
            pltpu.VMEM_SHARED((npad, d), jnp.float32),
            pltpu.VMEM((GROUP, CHUNK), jnp.int32),
            pltpu.VMEM((GROUP, CHUNK), jnp.int32),
            pltpu.VMEM((2, CHUNK, d), jnp.float32),
            pltpu.SemaphoreType.DMA((2,)),
        ],
    )
    def prop(x_hbm, src_hbm, dst_hbm, out_hbm, acc, sidx, didx, rows, gsem):
        cid = lax.axis_index("c")
        sid = lax.axis_index("s")
        wid = cid * NUM_SUBCORES + sid

        zero = jnp.zeros((LANES,), jnp.float32)

        def zfill(i, _):
            r = i // (d // LANES)
            col = (i % (d // LANES)) * LANES
            rows[0, r, pl.ds(col, LANES)] = zero
            return 0

        lax.fori_loop(0, ZROWS * (d // LANES), zfill, 0)
        zsrc = rows.at[0, pl.ds(0, ZROWS)]

        def zcopy(j, _):
            pltpu.sync_copy(
                zsrc, acc.at[pl.ds(sid * SUBROWS + j * ZROWS, ZROWS)])
            return 0

        lax.fori_loop(0, SUBROWS // ZROWS, zcopy, 0)

        @pl.when(sid == NUM_SUBCORES - 1)
        def _():
            def ztail(j, _):
                pltpu.sync_copy(
                    zsrc,
                    acc.at[pl.ds(NUM_SUBCORES * SUBROWS + j * ZROWS, ZROWS)])
                return 0
            lax.fori_loop(0, tail // ZROWS, ztail, 0)

        plsc.subcore_barrier()

        def group_body(g, _):
            pltpu.sync_copy(src_hbm.at[wid, g], sidx)
            pltpu.sync_copy(dst_hbm.at[wid, g], didx)
            pltpu.async_copy(x_hbm.at[sidx.at[0]], rows.at[0], gsem.at[0])

            def body(ci, _):
                b = lax.rem(ci, 2)
                nb = 1 - b

                @pl.when(ci + 1 < GROUP)
                def _():
                    pltpu.async_copy(x_hbm.at[sidx.at[ci + 1]], rows.at[nb],
                                     gsem.at[nb])

                pltpu.make_async_copy(x_hbm.at[sidx.at[ci]], rows.at[b],
                                      gsem.at[b]).wait()
                pltpu.sync_copy(rows.at[b], acc.at[didx.at[ci]], add=True)
                return 0

            lax.fori_loop(0, GROUP, body, 0)
            return 0

        lax.fori_loop(0, ngroups, group_body, 0)
        plsc.subcore_barrier()

        rbase = sid * SUBROWS
        pltpu.sync_copy(acc.at[pl.ds(rbase, SUBROWS)],
                        out_hbm.at[cid, pl.ds(rbase, SUBROWS)])

        @pl.when(sid == NUM_SUBCORES - 1)
        def _():
            tbase = NUM_SUBCORES * SUBROWS
            pltpu.sync_copy(acc.at[pl.ds(tbase, tail)],
                            out_hbm.at[cid, pl.ds(tbase, tail)])

    return prop(x, src4, dst4)


def _dense(p, w, block, final):
    n = p.shape[1]
    d = p.shape[2]

    def body(p_ref, w_ref, o_ref):
        h = p_ref[0] + p_ref[1]
        z = lax.dot_general(h, w_ref[...], (((1,), (1,)), ((), ())),
                            preferred_element_type=jnp.float32)
        if final:
            m = jnp.max(z, axis=1, keepdims=True)
            s = z - m
            lse = jnp.log(jnp.sum(jnp.exp(s), axis=1, keepdims=True))
            o_ref[...] = s - lse
        else:
            o_ref[...] = jnp.maximum(z, 0.0)

    return pl.pallas_call(
        body,
        grid=(n // block,),
        in_specs=[
            pl.BlockSpec((NUM_CORES, block, d), lambda i: (0, i, 0)),
            pl.BlockSpec((d, d), lambda i: (0, 0)),
        ],
        out_specs=pl.BlockSpec((block, d), lambda i: (i, 0)),
        out_shape=jax.ShapeDtypeStruct((n, d), jnp.float32),
    )(p, w)


def kernel(x, edge_index, W1, W2):
    src4, dst4 = _pad_edges(edge_index[0], edge_index[1], x.shape[0])
    p = _propagate(x, src4, dst4)
    h = _dense(p, W1, 1000, final=False)
    q = _propagate(h, src4, dst4)
    return _dense(q, W2, 1000, final=True)


# speedup vs baseline: 3.3895x; 1.7772x over previous
import functools

import jax
import jax.numpy as jnp
from jax import lax
from jax.experimental import pallas as pl
from jax.experimental.pallas import tpu as pltpu
from jax.experimental.pallas import tpu_sc as plsc

NUM_CORES = 2
NUM_SUBCORES = 16
LANES = 16

CHUNK = 80
GROUP = 25
ZROWS = 16
SUBROWS = 624
NWORKERS = NUM_CORES * NUM_SUBCORES


def _pad_edges(src, dst, n):
    e = src.shape[0]
    ngroups = -(-e // (NWORKERS * GROUP * CHUNK))
    pad = NWORKERS * ngroups * GROUP * CHUNK - e
    if pad:
        src = jnp.concatenate([src, jnp.zeros((pad,), jnp.int32)])
        dst = jnp.concatenate(
            [dst, n + (jnp.arange(pad, dtype=jnp.int32) % 8)])
    shape = (NWORKERS, ngroups, GROUP, CHUNK)
    return src.reshape(shape), dst.reshape(shape)


def _propagate(x, src4, dst4):
    n, d = x.shape
    ngroups = src4.shape[1]
    npad = n + 8
    tail = n - SUBROWS * NUM_SUBCORES

    mesh = plsc.VectorSubcoreMesh(core_axis_name="c", subcore_axis_name="s")

    @functools.partial(
        pl.kernel,
        mesh=mesh,
        out_type=jax.ShapeDtypeStruct((NUM_CORES, n, d), jnp.float32),
        scratch_types=[
            pltpu.VMEM_SHARED((npad, d), jnp.float32),
            pltpu.VMEM((GROUP, CHUNK), jnp.int32),
            pltpu.VMEM((GROUP, CHUNK), jnp.int32),
            pltpu.VMEM((2, CHUNK, d), jnp.float32),
            pltpu.SemaphoreType.DMA((2,)),
            pltpu.SemaphoreType.DMA((2,)),
        ],
    )
    def prop(x_hbm, src_hbm, dst_hbm, out_hbm, acc, sidx, didx, rows, gsem,
             ssem):
        cid = lax.axis_index("c")
        sid = lax.axis_index("s")
        wid = cid * NUM_SUBCORES + sid

        zero = jnp.zeros((LANES,), jnp.float32)

        def zfill(i, _):
            r = i // (d // LANES)
            col = (i % (d // LANES)) * LANES
            rows[0, r, pl.ds(col, LANES)] = zero
            return 0

        lax.fori_loop(0, ZROWS * (d // LANES), zfill, 0)
        zsrc = rows.at[0, pl.ds(0, ZROWS)]

        def zcopy(j, _):
            pltpu.sync_copy(
                zsrc, acc.at[pl.ds(sid * SUBROWS + j * ZROWS, ZROWS)])
            return 0

        lax.fori_loop(0, SUBROWS // ZROWS, zcopy, 0)

        @pl.when(sid == NUM_SUBCORES - 1)
        def _():
            def ztail(j, _):
                pltpu.sync_copy(
                    zsrc,
                    acc.at[pl.ds(NUM_SUBCORES * SUBROWS + j * ZROWS, ZROWS)])
                return 0
            lax.fori_loop(0, tail // ZROWS, ztail, 0)

        plsc.subcore_barrier()

        def group_body(g, _):
            pltpu.sync_copy(src_hbm.at[wid, g], sidx)
            pltpu.sync_copy(dst_hbm.at[wid, g], didx)
            pltpu.async_copy(x_hbm.at[sidx.at[0]], rows.at[0], gsem.at[0])

            def body(ci, _):
                b = lax.rem(ci, 2)
                nb = 1 - b

                @pl.when(jnp.logical_and(ci >= 1, ci + 1 < GROUP))
                def _():
                    pltpu.make_async_copy(rows.at[nb],
                                          acc.at[didx.at[ci - 1]],
                                          ssem.at[nb]).wait()

                @pl.when(ci + 1 < GROUP)
                def _():
                    pltpu.async_copy(x_hbm.at[sidx.at[ci + 1]], rows.at[nb],
                                     gsem.at[nb])

                pltpu.make_async_copy(x_hbm.at[sidx.at[ci]], rows.at[b],
                                      gsem.at[b]).wait()
                pltpu.async_copy(rows.at[b], acc.at[didx.at[ci]], ssem.at[b],
                                 add=True)
                return 0

            lax.fori_loop(0, GROUP, body, 0)
            lastb = (GROUP - 1) % 2
            pltpu.make_async_copy(rows.at[1 - lastb],
                                  acc.at[didx.at[GROUP - 2]],
                                  ssem.at[1 - lastb]).wait()
            pltpu.make_async_copy(rows.at[lastb], acc.at[didx.at[GROUP - 1]],
                                  ssem.at[lastb]).wait()
            return 0

        lax.fori_loop(0, ngroups, group_body, 0)
        plsc.subcore_barrier()

        rbase = sid * SUBROWS
        pltpu.sync_copy(acc.at[pl.ds(rbase, SUBROWS)],
                        out_hbm.at[cid, pl.ds(rbase, SUBROWS)])

        @pl.when(sid == NUM_SUBCORES - 1)
        def _():
            tbase = NUM_SUBCORES * SUBROWS
            pltpu.sync_copy(acc.at[pl.ds(tbase, tail)],
                            out_hbm.at[cid, pl.ds(tbase, tail)])

    return prop(x, src4, dst4)


def _dense(p, w, block, final):
    n = p.shape[1]
    d = p.shape[2]

    def body(p_ref, w_ref, o_ref):
        h = p_ref[0] + p_ref[1]
        z = lax.dot_general(h, w_ref[...], (((1,), (1,)), ((), ())),
                            preferred_element_type=jnp.float32)
        if final:
            m = jnp.max(z, axis=1, keepdims=True)
            s = z - m
            lse = jnp.log(jnp.sum(jnp.exp(s), axis=1, keepdims=True))
            o_ref[...] = s - lse
        else:
            o_ref[...] = jnp.maximum(z, 0.0)

    return pl.pallas_call(
        body,
        grid=(n // block,),
        in_specs=[
            pl.BlockSpec((NUM_CORES, block, d), lambda i: (0, i, 0)),
            pl.BlockSpec((d, d), lambda i: (0, 0)),
        ],
        out_specs=pl.BlockSpec((block, d), lambda i: (i, 0)),
        out_shape=jax.ShapeDtypeStruct((n, d), jnp.float32),
    )(p, w)


def kernel(x, edge_index, W1, W2):
    src4, dst4 = _pad_edges(edge_index[0], edge_index[1], x.shape[0])
    p = _propagate(x, src4, dst4)
    h = _dense(p, W1, 1000, final=False)
    q = _propagate(h, src4, dst4)
    return _dense(q, W2, 1000, final=True)


# speedup vs baseline: 3.5769x; 1.0553x over previous
import functools

import jax
import jax.numpy as jnp
from jax import lax
from jax.experimental import pallas as pl
from jax.experimental.pallas import tpu as pltpu
from jax.experimental.pallas import tpu_sc as plsc

NUM_CORES = 2
NUM_SUBCORES = 16
LANES = 16

CHUNK = 80
GROUP = 25
ZROWS = 16
SUBROWS = 624
NWORKERS = NUM_CORES * NUM_SUBCORES


def _pad_edges(src, dst, n):
    e = src.shape[0]
    ngroups = -(-e // (NWORKERS * GROUP * CHUNK))
    pad = NWORKERS * ngroups * GROUP * CHUNK - e
    if pad:
        src = jnp.concatenate([src, jnp.zeros((pad,), jnp.int32)])
        dst = jnp.concatenate(
            [dst, n + (jnp.arange(pad, dtype=jnp.int32) % 8)])
    shape = (NWORKERS, ngroups, GROUP, CHUNK)
    return src.reshape(shape), dst.reshape(shape)


def _propagate(x, src4, dst4):
    n, d = x.shape
    ngroups = src4.shape[1]
    npad = n + 8
    tail = n - SUBROWS * NUM_SUBCORES

    mesh = plsc.VectorSubcoreMesh(core_axis_name="c", subcore_axis_name="s")

    @functools.partial(
        pl.kernel,
        mesh=mesh,
        out_type=jax.ShapeDtypeStruct((NUM_CORES, n, d), jnp.float32),
        scratch_types=[
            pltpu.VMEM_SHARED((npad, d), jnp.float32),
            pltpu.VMEM((2, GROUP, CHUNK), jnp.int32),
            pltpu.VMEM((2, GROUP, CHUNK), jnp.int32),
            pltpu.VMEM((2, CHUNK, d), jnp.float32),
            pltpu.SemaphoreType.DMA((2,)),
            pltpu.SemaphoreType.DMA((2,)),
            pltpu.SemaphoreType.DMA((2,)),
        ],
    )
    def prop(x_hbm, src_hbm, dst_hbm, out_hbm, acc, sidx, didx, rows, gsem,
             ssem, isem):
        cid = lax.axis_index("c")
        sid = lax.axis_index("s")
        wid = cid * NUM_SUBCORES + sid

        zero = jnp.zeros((LANES,), jnp.float32)

        def zfill(i, _):
            r = i // (d // LANES)
            col = (i % (d // LANES)) * LANES
            rows[0, r, pl.ds(col, LANES)] = zero
            return 0

        lax.fori_loop(0, ZROWS * (d // LANES), zfill, 0)
        zsrc = rows.at[0, pl.ds(0, ZROWS)]

        def zcopy(j, _):
            pltpu.sync_copy(
                zsrc, acc.at[pl.ds(sid * SUBROWS + j * ZROWS, ZROWS)])
            return 0

        lax.fori_loop(0, SUBROWS // ZROWS, zcopy, 0)

        @pl.when(sid == NUM_SUBCORES - 1)
        def _():
            def ztail(j, _):
                pltpu.sync_copy(
                    zsrc,
                    acc.at[pl.ds(NUM_SUBCORES * SUBROWS + j * ZROWS, ZROWS)])
                return 0
            lax.fori_loop(0, tail // ZROWS, ztail, 0)

        pltpu.sync_copy(src_hbm.at[wid, 0], sidx.at[0])
        pltpu.sync_copy(dst_hbm.at[wid, 0], didx.at[0])
        plsc.subcore_barrier()

        nchunk = ngroups * GROUP

        def idx_of(ci):
            g = ci // GROUP
            return lax.rem(g, 2), ci - g * GROUP

        pltpu.async_copy(x_hbm.at[sidx.at[0, 0]], rows.at[0], gsem.at[0])

        def body(ci, _):
            b = lax.rem(ci, 2)
            nb = 1 - b
            g = ci // GROUP
            s = ci - g * GROUP
            ngb = 1 - lax.rem(g, 2)
            ib, is_ = idx_of(ci)

            @pl.when(jnp.logical_and(ci >= 1, ci + 1 < nchunk))
            def _():
                pib, pis = idx_of(ci - 1)
                pltpu.make_async_copy(rows.at[nb], acc.at[didx.at[pib, pis]],
                                      ssem.at[nb]).wait()

            @pl.when(jnp.logical_and(s == 0, (g + 1) * GROUP < nchunk))
            def _():
                pltpu.async_copy(src_hbm.at[wid, g + 1], sidx.at[ngb],
                                 isem.at[ngb])
                pltpu.async_copy(dst_hbm.at[wid, g + 1], didx.at[ngb],
                                 isem.at[ngb])

            @pl.when(jnp.logical_and(s == GROUP - 1, ci + 1 < nchunk))
            def _():
                pltpu.make_async_copy(src_hbm.at[wid, g + 1], sidx.at[ngb],
                                      isem.at[ngb]).wait()
                pltpu.make_async_copy(dst_hbm.at[wid, g + 1], didx.at[ngb],
                                      isem.at[ngb]).wait()

            @pl.when(ci + 1 < nchunk)
            def _():
                fib, fis = idx_of(ci + 1)
                pltpu.async_copy(x_hbm.at[sidx.at[fib, fis]], rows.at[nb],
                                 gsem.at[nb])

            pltpu.make_async_copy(x_hbm.at[sidx.at[ib, is_]], rows.at[b],
                                  gsem.at[b]).wait()
            pltpu.async_copy(rows.at[b], acc.at[didx.at[ib, is_]], ssem.at[b],
                             add=True)
            return 0

        lax.fori_loop(0, nchunk, body, 0)
        lb2, ls2 = (((nchunk - 2) // GROUP) % 2, (nchunk - 2) % GROUP)
        lb1, ls1 = (((nchunk - 1) // GROUP) % 2, (nchunk - 1) % GROUP)
        pltpu.make_async_copy(rows.at[nchunk % 2], acc.at[didx.at[lb2, ls2]],
                              ssem.at[nchunk % 2]).wait()
        pltpu.make_async_copy(rows.at[(nchunk - 1) % 2],
                              acc.at[didx.at[lb1, ls1]],
                              ssem.at[(nchunk - 1) % 2]).wait()
        plsc.subcore_barrier()

        rbase = sid * SUBROWS
        pltpu.sync_copy(acc.at[pl.ds(rbase, SUBROWS)],
                        out_hbm.at[cid, pl.ds(rbase, SUBROWS)])

        @pl.when(sid == NUM_SUBCORES - 1)
        def _():
            tbase = NUM_SUBCORES * SUBROWS
            pltpu.sync_copy(acc.at[pl.ds(tbase, tail)],
                            out_hbm.at[cid, pl.ds(tbase, tail)])

    return prop(x, src4, dst4)


def _dense(p, w, block, final):
    n = p.shape[1]
    d = p.shape[2]

    def body(p_ref, w_ref, o_ref):
        h = p_ref[0] + p_ref[1]
        z = lax.dot_general(h, w_ref[...], (((1,), (1,)), ((), ())),
                            preferred_element_type=jnp.float32)
        if final:
            m = jnp.max(z, axis=1, keepdims=True)
            s = z - m
            lse = jnp.log(jnp.sum(jnp.exp(s), axis=1, keepdims=True))
            o_ref[...] = s - lse
        else:
            o_ref[...] = jnp.maximum(z, 0.0)

    return pl.pallas_call(
        body,
        grid=(n // block,),
        in_specs=[
            pl.BlockSpec((NUM_CORES, block, d), lambda i: (0, i, 0)),
            pl.BlockSpec((d, d), lambda i: (0, 0)),
        ],
        out_specs=pl.BlockSpec((block, d), lambda i: (i, 0)),
        out_shape=jax.ShapeDtypeStruct((n, d), jnp.float32),
    )(p, w)


def kernel(x, edge_index, W1, W2):
    src4, dst4 = _pad_edges(edge_index[0], edge_index[1], x.shape[0])
    p = _propagate(x, src4, dst4)
    h = _dense(p, W1, 1000, final=False)
    q = _propagate(h, src4, dst4)
    return _dense(q, W2, 1000, final=True)
